# baseline (speedup 1.0000x reference)
import jax
import jax.numpy as jnp
from jax import lax
from jax.experimental import pallas as pl
from jax.experimental.pallas import tpu as pltpu
from jax.experimental.pallas import tpu_sc as plsc

_N = 10000
_E = 320000
_D = 128
_NPAD = 10240

_NC, _NS, _L = 2, 16, 16
_SLICE = _NPAD // _NS
_EC_SCAL = _E // _NS
_EC_ROW = _E // (_NC * _NS)
_SB = 2000
_RB = 80
_RNB = _EC_ROW // _RB
_RING = 4
_PF = 2

_RBLK = 1024
_GRID = _NPAD // _RBLK


def _dense_body(x_ref, w1_ref, w2_ref, wp_ref, z_ref, p_ref):
    dn = (((1,), (1,)), ((), ()))
    z1 = jnp.maximum(
        lax.dot_general(x_ref[...], w1_ref[...], dn,
                        preferred_element_type=jnp.float32), 0.0)
    z = jnp.maximum(
        lax.dot_general(z1, w2_ref[...], dn,
                        preferred_element_type=jnp.float32), 0.0)
    z_ref[...] = z
    p_ref[...] = lax.dot_general(wp_ref[...], z, dn,
                                 preferred_element_type=jnp.float32)


def _dense(x_pad, w1, w2, wp):
    return pl.pallas_call(
        _dense_body,
        grid=(_GRID,),
        in_specs=[
            pl.BlockSpec((_RBLK, _D), lambda i: (i, 0)),
            pl.BlockSpec((_D, _D), lambda i: (0, 0)),
            pl.BlockSpec((_D, _D), lambda i: (0, 0)),
            pl.BlockSpec((8, _D), lambda i: (0, 0)),
        ],
        out_specs=[
            pl.BlockSpec((_RBLK, _D), lambda i: (i, 0)),
            pl.BlockSpec((8, _RBLK), lambda i: (0, i)),
        ],
        out_shape=[
            jax.ShapeDtypeStruct((_NPAD, _D), jnp.float32),
            jax.ShapeDtypeStruct((8, _NPAD), jnp.float32),
        ],
    )(x_pad, w1, w2, wp)


def _leaky(v):
    return jnp.where(v >= 0, v, 0.01 * v)


_GDN = lax.GatherDimensionNumbers(
    offset_dims=(), collapsed_slice_dims=(0,), start_index_map=(0,))


def _take16(v, idx):
    return lax.gather(v, idx[:, None], _GDN, slice_sizes=(1,),
                      mode=lax.GatherScatterMode.PROMISE_IN_BOUNDS)


def _segscan(dk, v, iota, is_add):
    for k in (1, 2, 4, 8):
        idx = jnp.maximum(iota - k, 0)
        vs = _take16(v, idx)
        dsft = _take16(dk, idx)
        ok = (dsft == dk) & (iota >= k)
        if is_add:
            v = v + jnp.where(ok, vs, 0.0)
        else:
            v = jnp.where(ok, jnp.maximum(v, vs), v)
    return v


def _last_mask(dk, iota):
    nxt = _take16(dk, jnp.minimum(iota + 1, _L - 1))
    return (nxt != dk) | (iota == _L - 1)


def _sc_scalar_body(p8_hbm, dst_hbm, src_hbm, s_hbm, ex_hbm,
                    pdst_v, psrc_v, m_v, sloc_v, red_v,
                    dstb_a, dstb_b, srcb_a, srcb_b,
                    exb_v, slice_v,
                    red_sh, m_sh, stp0, stp1):
    cid = lax.axis_index("c")
    tid = lax.axis_index("s")
    nbase = tid * _SLICE
    iota = lax.iota(jnp.int32, _L)
    stp = (stp0, stp1)
    dstb = (dstb_a, dstb_b)
    srcb = (srcb_a, srcb_b)
    _NEB = _EC_SCAL // _SB

    def _est_issue(b, par):
        eb = tid * _EC_SCAL + b * _SB
        pltpu.async_copy(dst_hbm.at[pl.ds(eb, _SB)], dstb[par], stp[par])
        pltpu.async_copy(src_hbm.at[pl.ds(eb, _SB)], srcb[par], stp[par])

    def _est_wait(b, par):
        eb = tid * _EC_SCAL + b * _SB
        pltpu.make_async_copy(dst_hbm.at[pl.ds(eb, _SB)], dstb[par],
                              stp[par]).wait()
        pltpu.make_async_copy(src_hbm.at[pl.ds(eb, _SB)], srcb[par],
                              stp[par]).wait()

    pltpu.sync_copy(p8_hbm.at[0], pdst_v)
    pltpu.sync_copy(p8_hbm.at[1], psrc_v)

    neg = jnp.full((_L,), -1e30, jnp.float32)
    zv = jnp.zeros((_L,), jnp.float32)

    def _init(i, c):
        m_v[pl.ds(i * _L, _L)] = neg
        sloc_v[pl.ds(i * _L, _L)] = zv
        return c

    lax.fori_loop(0, _NPAD // _L, _init, 0)

    _est_issue(0, 0)

    def _emax_block(g, c):
      for par in range(2):
        b = g * 2 + par
        _est_wait(b, par)

        @pl.when(b + 1 < _NEB)
        def _():
            _est_issue(b + 1, 1 - par)

        def _vstep(j, cc):
            sl = pl.ds(j * _L, _L)
            d = dstb[par][sl]
            si = srcb[par][sl]
            e = _leaky(plsc.load_gather(pdst_v, [d]) +
                       plsc.load_gather(psrc_v, [si]))
            dk, val = plsc.sort_key_val(d, e)
            val = _segscan(dk, val, iota, is_add=False)
            last = _last_mask(dk, iota)
            cur = plsc.load_gather(m_v, [dk])
            plsc.store_scatter(m_v, [dk], jnp.maximum(cur, val), mask=last)
            return cc

        lax.fori_loop(0, _SB // _L, _vstep, 0, unroll=4)
      return c

    lax.fori_loop(0, _NEB // 2, _emax_block, 0)

    pltpu.sync_copy(m_v, red_sh.at[tid])
    plsc.subcore_barrier()
    for r in range(_NS):
        pltpu.sync_copy(red_sh.at[r, pl.ds(nbase, _SLICE)], red_v.at[r])

    def _redmax(j, c):
        sl = pl.ds(j * _L, _L)
        acc = red_v[0, sl]
        for r in range(1, _NS):
            acc = jnp.maximum(acc, red_v[r, sl])
        slice_v[sl] = acc
        return c

    lax.fori_loop(0, _SLICE // _L, _redmax, 0)
    pltpu.sync_copy(slice_v, m_sh.at[pl.ds(nbase, _SLICE)])
    plsc.subcore_barrier()
    pltpu.sync_copy(m_sh, m_v)

    _est_issue(0, 0)

    def _ssum_block(g, c):
      for par in range(2):
        b = g * 2 + par
        ebase = tid * _EC_SCAL + b * _SB
        _est_wait(b, par)

        @pl.when(b + 1 < _NEB)
        def _():
            _est_issue(b + 1, 1 - par)

        def _vstep(j, cc):
            sl = pl.ds(j * _L, _L)
            d = dstb[par][sl]
            si = srcb[par][sl]
            e = _leaky(plsc.load_gather(pdst_v, [d]) +
                       plsc.load_gather(psrc_v, [si]))
            ex = jnp.exp(e - plsc.load_gather(m_v, [d]))
            exb_v[sl] = ex
            plsc.addupdate_scatter(sloc_v, [d], ex)
            return cc

        lax.fori_loop(0, _SB // _L, _vstep, 0, unroll=4)

        @pl.when((b < _NEB // 2) == (cid == 0))
        def _():
            pltpu.sync_copy(exb_v, ex_hbm.at[pl.ds(ebase, _SB)])

      return c

    lax.fori_loop(0, _NEB // 2, _ssum_block, 0)

    plsc.subcore_barrier()
    pltpu.sync_copy(sloc_v, red_sh.at[tid])
    plsc.subcore_barrier()
    for r in range(_NS):
        pltpu.sync_copy(red_sh.at[r, pl.ds(nbase, _SLICE)], red_v.at[r])

    def _redsum(j, c):
        sl = pl.ds(j * _L, _L)
        acc = red_v[0, sl]
        for r in range(1, _NS):
            acc = acc + red_v[r, sl]
        slice_v[sl] = acc
        return c

    lax.fori_loop(0, _SLICE // _L, _redsum, 0)

    @pl.when(cid == 0)
    def _():
        pltpu.sync_copy(slice_v, s_hbm.at[pl.ds(nbase, _SLICE)])


def _sc_scalar(p8, dst, src):
    mesh = plsc.VectorSubcoreMesh(core_axis_name="c", subcore_axis_name="s",
                                  num_cores=_NC, num_subcores=_NS)
    fn = pl.kernel(
        _sc_scalar_body,
        out_type=[
            jax.ShapeDtypeStruct((_NPAD,), jnp.float32),
            jax.ShapeDtypeStruct((_E,), jnp.float32),
        ],
        mesh=mesh,
        compiler_params=pltpu.CompilerParams(needs_layout_passes=False),
        scratch_types=[
            pltpu.VMEM((_NPAD,), jnp.float32),
            pltpu.VMEM((_NPAD,), jnp.float32),
            pltpu.VMEM((_NPAD,), jnp.float32),
            pltpu.VMEM((_NPAD,), jnp.float32),
            pltpu.VMEM((_NS, _SLICE), jnp.float32),
            pltpu.VMEM((_SB,), jnp.int32),
            pltpu.VMEM((_SB,), jnp.int32),
            pltpu.VMEM((_SB,), jnp.int32),
            pltpu.VMEM((_SB,), jnp.int32),
            pltpu.VMEM((_SB,), jnp.float32),
            pltpu.VMEM((_SLICE,), jnp.float32),
            pltpu.VMEM_SHARED((_NS, _NPAD), jnp.float32),
            pltpu.VMEM_SHARED((_NPAD,), jnp.float32),
            pltpu.SemaphoreType.DMA,
            pltpu.SemaphoreType.DMA,
        ],
    )
    return fn(p8, dst, src)


_IR = 8


def _sc_rows_body(zp_hbm, dst_hbm, src_hbm, ex_hbm, res_hbm,
                  ridx8, gidx8, exb8, rows3, res_sh,
                  sg0, sg1, sg2, sg3, ss0, ss1, ss2, ss3,
                  st0, st1, st2, st3, st4, st5, st6, st7):
    cid = lax.axis_index("c")
    tid = lax.axis_index("s")
    nbase = tid * _SLICE
    ebase0 = cid * (_E // _NC) + tid * _EC_ROW
    sgs = (sg0, sg1, sg2, sg3)
    sss = (ss0, ss1, ss2, ss3)
    sts = (st0, st1, st2, st3, st4, st5, st6, st7)

    zv = jnp.zeros((_L,), jnp.float32)

    def _zr(i, c):
        for q in range(_D // _L):
            rows3[0, i, pl.ds(q * _L, _L)] = zv
        return c

    lax.fori_loop(0, _RB, _zr, 0)
    for k in range(_SLICE // _RB):
        pltpu.sync_copy(rows3.at[0], res_sh.at[pl.ds(nbase + k * _RB, _RB)])
    plsc.subcore_barrier()

    def _stage_issue(b, s8):
        off = ebase0 + b * _RB
        pltpu.async_copy(dst_hbm.at[pl.ds(off, _RB)], ridx8.at[s8], sts[s8])
        pltpu.async_copy(src_hbm.at[pl.ds(off, _RB)], gidx8.at[s8], sts[s8])
        pltpu.async_copy(ex_hbm.at[pl.ds(off, _RB)], exb8.at[s8], sts[s8])

    def _stage_wait(b, s8):
        off = ebase0 + b * _RB
        pltpu.make_async_copy(dst_hbm.at[pl.ds(off, _RB)], ridx8.at[s8],
                              sts[s8]).wait()
        pltpu.make_async_copy(src_hbm.at[pl.ds(off, _RB)], gidx8.at[s8],
                              sts[s8]).wait()
        pltpu.make_async_copy(ex_hbm.at[pl.ds(off, _RB)], exb8.at[s8],
                              sts[s8]).wait()

    def _gather_issue(s8, r4):
        pltpu.async_copy(zp_hbm.at[gidx8.at[s8]], rows3.at[r4], sgs[r4])

    def _gather_wait(s8, r4):
        pltpu.make_async_copy(zp_hbm.at[gidx8.at[s8]], rows3.at[r4],
                              sgs[r4]).wait()

    def _scatter_issue(s8, r4):
        pltpu.async_copy(rows3.at[r4], res_sh.at[ridx8.at[s8]], sss[r4],
                         add=True)

    def _scatter_wait(s8, r4):
        pltpu.make_async_copy(rows3.at[r4], res_sh.at[ridx8.at[s8]],
                              sss[r4]).wait()

    for b in range(4):
        _stage_issue(b, b)
    for b in range(2):
        _stage_wait(b, b)
        _gather_issue(b, b)

    def _group(g, c):
        for i in range(_IR):
            b = g * _IR + i
            r4 = i % 4

            @pl.when(b < _RNB)
            def _():
                _gather_wait(i, r4)

                def _scale(gg, cc):
                    w16 = exb8[i, pl.ds(gg * _L, _L)]
                    for rr in range(_L):
                        row = gg * _L + rr
                        wsp = jnp.full((_L,), w16[rr], jnp.float32)
                        for q in range(_D // _L):
                            sl = pl.ds(q * _L, _L)
                            rows3[r4, row, sl] = rows3[r4, row, sl] * wsp
                    return cc

                lax.fori_loop(0, _RB // _L, _scale, 0)

                _scatter_issue(i, r4)

                q2 = b + 2

                @pl.when(q2 < _RNB)
                def _():
                    @pl.when(q2 >= 4)
                    def _():
                        _scatter_wait((i + 6) % _IR, (i + 2) % 4)

                    _stage_wait(q2, (i + 2) % _IR)
                    _gather_issue((i + 2) % _IR, (i + 2) % 4)

                q4 = b + 4

                @pl.when(q4 < _RNB)
                def _():
                    _stage_issue(q4, (i + 4) % _IR)

        return c

    lax.fori_loop(0, (_RNB + _IR - 1) // _IR, _group, 0)

    for b in range(_RNB - 4, _RNB):
        _scatter_wait(b % _IR, b % 4)

    plsc.subcore_barrier()
    pltpu.sync_copy(res_sh.at[pl.ds(nbase, _SLICE)],
                    res_hbm.at[cid, pl.ds(nbase, _SLICE)])


def _sc_rows(z_pad, dst, src, ex):
    mesh = plsc.VectorSubcoreMesh(core_axis_name="c", subcore_axis_name="s",
                                  num_cores=_NC, num_subcores=_NS)
    fn = pl.kernel(
        _sc_rows_body,
        out_type=jax.ShapeDtypeStruct((_NC, _NPAD, _D), jnp.float32),
        mesh=mesh,
        compiler_params=pltpu.CompilerParams(needs_layout_passes=False),
        scratch_types=[
            pltpu.VMEM((_IR, _RB), jnp.int32),
            pltpu.VMEM((_IR, _RB), jnp.int32),
            pltpu.VMEM((_IR, _RB), jnp.float32),
            pltpu.VMEM((_RING, _RB, _D), jnp.float32),
            pltpu.VMEM_SHARED((_NPAD, _D), jnp.float32),
        ] + [pltpu.SemaphoreType.DMA] * 16,
    )
    return fn(z_pad, dst, src, ex)


def _final_body(z_ref, r_ref, s_ref, h_ref):
    r = r_ref[...]
    s = s_ref[...]
    inv = 1.0 / jnp.where(s > 0, s, 1.0)
    h_ref[...] = jnp.maximum(z_ref[...] - (r[0] + r[1]) * inv, 0.0)


def _final(z_pad, res, s2d):
    return pl.pallas_call(
        _final_body,
        grid=(_GRID,),
        in_specs=[
            pl.BlockSpec((_RBLK, _D), lambda i: (i, 0)),
            pl.BlockSpec((_NC, _RBLK, _D), lambda i: (0, i, 0)),
            pl.BlockSpec((_RBLK, 1), lambda i: (i, 0)),
        ],
        out_specs=pl.BlockSpec((_RBLK, _D), lambda i: (i, 0)),
        out_shape=jax.ShapeDtypeStruct((_N, _D), jnp.float32),
    )(z_pad, res, s2d)


@jax.jit
def kernel(x, edge_index, W1, W2, a):
    x = x.astype(jnp.float32)
    src = edge_index[0].astype(jnp.int32)
    dst = edge_index[1].astype(jnp.int32)
    av = a[:, 0].astype(jnp.float32)
    wp = jnp.zeros((8, _D), jnp.float32)
    wp = wp.at[0].set(av[:_D]).at[1].set(av[_D:])
    z_pad, p8 = _dense(x, W1, W2, wp)
    s, ex = _sc_scalar(p8, dst, src)
    res = _sc_rows(z_pad, dst, src, ex)
    return _final(z_pad, res, s[:, None])


# speedup vs baseline: 1.0046x; 1.0046x over previous
import jax
import jax.numpy as jnp
from jax import lax
from jax.experimental import pallas as pl
from jax.experimental.pallas import tpu as pltpu
from jax.experimental.pallas import tpu_sc as plsc

_N = 10000
_E = 320000
_D = 128
_NPAD = 10240

_NC, _NS, _L = 2, 16, 16
_SLICE = _NPAD // _NS
_EC_SCAL = _E // _NS
_EC_ROW = _E // (_NC * _NS)
_SB = 2000
_RB = 80
_RNB = _EC_ROW // _RB
_RING = 4
_PF = 2

_RBLK = 1024
_GRID = _NPAD // _RBLK


def _dense_body(x_ref, w1_ref, w2_ref, wp_ref, z_ref, p_ref):
    dn = (((1,), (1,)), ((), ()))
    z1 = jnp.maximum(
        lax.dot_general(x_ref[...], w1_ref[...], dn,
                        preferred_element_type=jnp.float32), 0.0)
    z = jnp.maximum(
        lax.dot_general(z1, w2_ref[...], dn,
                        preferred_element_type=jnp.float32), 0.0)
    z_ref[...] = z
    p_ref[...] = lax.dot_general(wp_ref[...], z, dn,
                                 preferred_element_type=jnp.float32)


def _dense(x_pad, w1, w2, wp):
    return pl.pallas_call(
        _dense_body,
        grid=(_GRID,),
        in_specs=[
            pl.BlockSpec((_RBLK, _D), lambda i: (i, 0)),
            pl.BlockSpec((_D, _D), lambda i: (0, 0)),
            pl.BlockSpec((_D, _D), lambda i: (0, 0)),
            pl.BlockSpec((8, _D), lambda i: (0, 0)),
        ],
        out_specs=[
            pl.BlockSpec((_RBLK, _D), lambda i: (i, 0)),
            pl.BlockSpec((8, _RBLK), lambda i: (0, i)),
        ],
        out_shape=[
            jax.ShapeDtypeStruct((_NPAD, _D), jnp.float32),
            jax.ShapeDtypeStruct((8, _NPAD), jnp.float32),
        ],
    )(x_pad, w1, w2, wp)


def _leaky(v):
    return jnp.where(v >= 0, v, 0.01 * v)


_GDN = lax.GatherDimensionNumbers(
    offset_dims=(), collapsed_slice_dims=(0,), start_index_map=(0,))


def _take16(v, idx):
    return lax.gather(v, idx[:, None], _GDN, slice_sizes=(1,),
                      mode=lax.GatherScatterMode.PROMISE_IN_BOUNDS)


def _segscan(dk, v, iota, is_add):
    for k in (1, 2, 4, 8):
        idx = jnp.maximum(iota - k, 0)
        vs = _take16(v, idx)
        dsft = _take16(dk, idx)
        ok = (dsft == dk) & (iota >= k)
        if is_add:
            v = v + jnp.where(ok, vs, 0.0)
        else:
            v = jnp.where(ok, jnp.maximum(v, vs), v)
    return v


def _last_mask(dk, iota):
    nxt = _take16(dk, jnp.minimum(iota + 1, _L - 1))
    return (nxt != dk) | (iota == _L - 1)


def _sc_scalar_body(p8_hbm, dst_hbm, src_hbm, s_hbm, ex_hbm,
                    pdst_v, psrc_v, m_v, sloc_v, red_v,
                    dstb_a, dstb_b, srcb_a, srcb_b,
                    exb_v, slice_v,
                    red_sh, m_sh, stp0, stp1):
    cid = lax.axis_index("c")
    tid = lax.axis_index("s")
    nbase = tid * _SLICE
    iota = lax.iota(jnp.int32, _L)
    stp = (stp0, stp1)
    dstb = (dstb_a, dstb_b)
    srcb = (srcb_a, srcb_b)
    _NEB = _EC_SCAL // _SB

    def _est_issue(b, par):
        eb = tid * _EC_SCAL + b * _SB
        pltpu.async_copy(dst_hbm.at[pl.ds(eb, _SB)], dstb[par], stp[par])
        pltpu.async_copy(src_hbm.at[pl.ds(eb, _SB)], srcb[par], stp[par])

    def _est_wait(b, par):
        eb = tid * _EC_SCAL + b * _SB
        pltpu.make_async_copy(dst_hbm.at[pl.ds(eb, _SB)], dstb[par],
                              stp[par]).wait()
        pltpu.make_async_copy(src_hbm.at[pl.ds(eb, _SB)], srcb[par],
                              stp[par]).wait()

    pltpu.sync_copy(p8_hbm.at[0], pdst_v)
    pltpu.sync_copy(p8_hbm.at[1], psrc_v)

    neg = jnp.full((_L,), -1e30, jnp.float32)
    zv = jnp.zeros((_L,), jnp.float32)

    def _init(i, c):
        m_v[pl.ds(i * _L, _L)] = neg
        sloc_v[pl.ds(i * _L, _L)] = zv
        return c

    lax.fori_loop(0, _NPAD // _L, _init, 0)

    _est_issue(0, 0)

    def _emax_block(g, c):
      for par in range(2):
        b = g * 2 + par
        _est_wait(b, par)

        @pl.when(b + 1 < _NEB)
        def _():
            _est_issue(b + 1, 1 - par)

        def _vstep(j, cc):
            sl = pl.ds(j * _L, _L)
            d = dstb[par][sl]
            si = srcb[par][sl]
            e = _leaky(plsc.load_gather(pdst_v, [d]) +
                       plsc.load_gather(psrc_v, [si]))
            dk, val = plsc.sort_key_val(d, e)
            val = _segscan(dk, val, iota, is_add=False)
            last = _last_mask(dk, iota)
            cur = plsc.load_gather(m_v, [dk])
            plsc.store_scatter(m_v, [dk], jnp.maximum(cur, val), mask=last)
            return cc

        lax.fori_loop(0, _SB // _L, _vstep, 0, unroll=2)
      return c

    lax.fori_loop(0, _NEB // 2, _emax_block, 0)

    pltpu.sync_copy(m_v, red_sh.at[tid])
    plsc.subcore_barrier()
    for r in range(_NS):
        pltpu.sync_copy(red_sh.at[r, pl.ds(nbase, _SLICE)], red_v.at[r])

    def _redmax(j, c):
        sl = pl.ds(j * _L, _L)
        acc = red_v[0, sl]
        for r in range(1, _NS):
            acc = jnp.maximum(acc, red_v[r, sl])
        slice_v[sl] = acc
        return c

    lax.fori_loop(0, _SLICE // _L, _redmax, 0)
    pltpu.sync_copy(slice_v, m_sh.at[pl.ds(nbase, _SLICE)])
    plsc.subcore_barrier()
    pltpu.sync_copy(m_sh, m_v)

    _est_issue(0, 0)

    def _ssum_block(g, c):
      for par in range(2):
        b = g * 2 + par
        ebase = tid * _EC_SCAL + b * _SB
        _est_wait(b, par)

        @pl.when(b + 1 < _NEB)
        def _():
            _est_issue(b + 1, 1 - par)

        def _vstep(j, cc):
            sl = pl.ds(j * _L, _L)
            d = dstb[par][sl]
            si = srcb[par][sl]
            e = _leaky(plsc.load_gather(pdst_v, [d]) +
                       plsc.load_gather(psrc_v, [si]))
            ex = jnp.exp(e - plsc.load_gather(m_v, [d]))
            exb_v[sl] = ex
            plsc.addupdate_scatter(sloc_v, [d], ex)
            return cc

        lax.fori_loop(0, _SB // _L, _vstep, 0, unroll=2)

        @pl.when((b < _NEB // 2) == (cid == 0))
        def _():
            pltpu.sync_copy(exb_v, ex_hbm.at[pl.ds(ebase, _SB)])

      return c

    lax.fori_loop(0, _NEB // 2, _ssum_block, 0)

    plsc.subcore_barrier()
    pltpu.sync_copy(sloc_v, red_sh.at[tid])
    plsc.subcore_barrier()
    for r in range(_NS):
        pltpu.sync_copy(red_sh.at[r, pl.ds(nbase, _SLICE)], red_v.at[r])

    def _redsum(j, c):
        sl = pl.ds(j * _L, _L)
        acc = red_v[0, sl]
        for r in range(1, _NS):
            acc = acc + red_v[r, sl]
        slice_v[sl] = acc
        return c

    lax.fori_loop(0, _SLICE // _L, _redsum, 0)

    @pl.when(cid == 0)
    def _():
        pltpu.sync_copy(slice_v, s_hbm.at[pl.ds(nbase, _SLICE)])


def _sc_scalar(p8, dst, src):
    mesh = plsc.VectorSubcoreMesh(core_axis_name="c", subcore_axis_name="s",
                                  num_cores=_NC, num_subcores=_NS)
    fn = pl.kernel(
        _sc_scalar_body,
        out_type=[
            jax.ShapeDtypeStruct((_NPAD,), jnp.float32),
            jax.ShapeDtypeStruct((_E,), jnp.float32),
        ],
        mesh=mesh,
        compiler_params=pltpu.CompilerParams(needs_layout_passes=False),
        scratch_types=[
            pltpu.VMEM((_NPAD,), jnp.float32),
            pltpu.VMEM((_NPAD,), jnp.float32),
            pltpu.VMEM((_NPAD,), jnp.float32),
            pltpu.VMEM((_NPAD,), jnp.float32),
            pltpu.VMEM((_NS, _SLICE), jnp.float32),
            pltpu.VMEM((_SB,), jnp.int32),
            pltpu.VMEM((_SB,), jnp.int32),
            pltpu.VMEM((_SB,), jnp.int32),
            pltpu.VMEM((_SB,), jnp.int32),
            pltpu.VMEM((_SB,), jnp.float32),
            pltpu.VMEM((_SLICE,), jnp.float32),
            pltpu.VMEM_SHARED((_NS, _NPAD), jnp.float32),
            pltpu.VMEM_SHARED((_NPAD,), jnp.float32),
            pltpu.SemaphoreType.DMA,
            pltpu.SemaphoreType.DMA,
        ],
    )
    return fn(p8, dst, src)


_IR = 8


def _sc_rows_body(zp_hbm, dst_hbm, src_hbm, ex_hbm, res_hbm,
                  ridx8, gidx8, exb8, rows3, res_sh,
                  sg0, sg1, sg2, sg3, ss0, ss1, ss2, ss3,
                  st0, st1, st2, st3, st4, st5, st6, st7):
    cid = lax.axis_index("c")
    tid = lax.axis_index("s")
    nbase = tid * _SLICE
    ebase0 = cid * (_E // _NC) + tid * _EC_ROW
    sgs = (sg0, sg1, sg2, sg3)
    sss = (ss0, ss1, ss2, ss3)
    sts = (st0, st1, st2, st3, st4, st5, st6, st7)

    zv = jnp.zeros((_L,), jnp.float32)

    def _zr(i, c):
        for q in range(_D // _L):
            rows3[0, i, pl.ds(q * _L, _L)] = zv
        return c

    lax.fori_loop(0, _RB, _zr, 0)
    for k in range(_SLICE // _RB):
        pltpu.sync_copy(rows3.at[0], res_sh.at[pl.ds(nbase + k * _RB, _RB)])
    plsc.subcore_barrier()

    def _stage_issue(b, s8):
        off = ebase0 + b * _RB
        pltpu.async_copy(dst_hbm.at[pl.ds(off, _RB)], ridx8.at[s8], sts[s8])
        pltpu.async_copy(src_hbm.at[pl.ds(off, _RB)], gidx8.at[s8], sts[s8])
        pltpu.async_copy(ex_hbm.at[pl.ds(off, _RB)], exb8.at[s8], sts[s8])

    def _stage_wait(b, s8):
        off = ebase0 + b * _RB
        pltpu.make_async_copy(dst_hbm.at[pl.ds(off, _RB)], ridx8.at[s8],
                              sts[s8]).wait()
        pltpu.make_async_copy(src_hbm.at[pl.ds(off, _RB)], gidx8.at[s8],
                              sts[s8]).wait()
        pltpu.make_async_copy(ex_hbm.at[pl.ds(off, _RB)], exb8.at[s8],
                              sts[s8]).wait()

    def _gather_issue(s8, r4):
        pltpu.async_copy(zp_hbm.at[gidx8.at[s8]], rows3.at[r4], sgs[r4])

    def _gather_wait(s8, r4):
        pltpu.make_async_copy(zp_hbm.at[gidx8.at[s8]], rows3.at[r4],
                              sgs[r4]).wait()

    def _scatter_issue(s8, r4):
        pltpu.async_copy(rows3.at[r4], res_sh.at[ridx8.at[s8]], sss[r4],
                         add=True)

    def _scatter_wait(s8, r4):
        pltpu.make_async_copy(rows3.at[r4], res_sh.at[ridx8.at[s8]],
                              sss[r4]).wait()

    for b in range(4):
        _stage_issue(b, b)
    for b in range(2):
        _stage_wait(b, b)
        _gather_issue(b, b)

    def _group(g, c):
        for i in range(_IR):
            b = g * _IR + i
            r4 = i % 4

            @pl.when(b < _RNB)
            def _():
                _gather_wait(i, r4)

                def _scale(gg, cc):
                    w16 = exb8[i, pl.ds(gg * _L, _L)]
                    for rr in range(_L):
                        row = gg * _L + rr
                        wsp = jnp.full((_L,), w16[rr], jnp.float32)
                        for q in range(_D // _L):
                            sl = pl.ds(q * _L, _L)
                            rows3[r4, row, sl] = rows3[r4, row, sl] * wsp
                    return cc

                lax.fori_loop(0, _RB // _L, _scale, 0)

                _scatter_issue(i, r4)

                q2 = b + 2

                @pl.when(q2 < _RNB)
                def _():
                    @pl.when(q2 >= 4)
                    def _():
                        _scatter_wait((i + 6) % _IR, (i + 2) % 4)

                    _stage_wait(q2, (i + 2) % _IR)
                    _gather_issue((i + 2) % _IR, (i + 2) % 4)

                q4 = b + 4

                @pl.when(q4 < _RNB)
                def _():
                    _stage_issue(q4, (i + 4) % _IR)

        return c

    lax.fori_loop(0, (_RNB + _IR - 1) // _IR, _group, 0)

    for b in range(_RNB - 4, _RNB):
        _scatter_wait(b % _IR, b % 4)

    plsc.subcore_barrier()
    pltpu.sync_copy(res_sh.at[pl.ds(nbase, _SLICE)],
                    res_hbm.at[cid, pl.ds(nbase, _SLICE)])


def _sc_rows(z_pad, dst, src, ex):
    mesh = plsc.VectorSubcoreMesh(core_axis_name="c", subcore_axis_name="s",
                                  num_cores=_NC, num_subcores=_NS)
    fn = pl.kernel(
        _sc_rows_body,
        out_type=jax.ShapeDtypeStruct((_NC, _NPAD, _D), jnp.float32),
        mesh=mesh,
        compiler_params=pltpu.CompilerParams(needs_layout_passes=False),
        scratch_types=[
            pltpu.VMEM((_IR, _RB), jnp.int32),
            pltpu.VMEM((_IR, _RB), jnp.int32),
            pltpu.VMEM((_IR, _RB), jnp.float32),
            pltpu.VMEM((_RING, _RB, _D), jnp.float32),
            pltpu.VMEM_SHARED((_NPAD, _D), jnp.float32),
        ] + [pltpu.SemaphoreType.DMA] * 16,
    )
    return fn(z_pad, dst, src, ex)


def _final_body(z_ref, r_ref, s_ref, h_ref):
    r = r_ref[...]
    s = s_ref[...]
    inv = 1.0 / jnp.where(s > 0, s, 1.0)
    h_ref[...] = jnp.maximum(z_ref[...] - (r[0] + r[1]) * inv, 0.0)


def _final(z_pad, res, s2d):
    return pl.pallas_call(
        _final_body,
        grid=(_GRID,),
        in_specs=[
            pl.BlockSpec((_RBLK, _D), lambda i: (i, 0)),
            pl.BlockSpec((_NC, _RBLK, _D), lambda i: (0, i, 0)),
            pl.BlockSpec((_RBLK, 1), lambda i: (i, 0)),
        ],
        out_specs=pl.BlockSpec((_RBLK, _D), lambda i: (i, 0)),
        out_shape=jax.ShapeDtypeStruct((_N, _D), jnp.float32),
    )(z_pad, res, s2d)


@jax.jit
def kernel(x, edge_index, W1, W2, a):
    x = x.astype(jnp.float32)
    src = edge_index[0].astype(jnp.int32)
    dst = edge_index[1].astype(jnp.int32)
    av = a[:, 0].astype(jnp.float32)
    wp = jnp.zeros((8, _D), jnp.float32)
    wp = wp.at[0].set(av[:_D]).at[1].set(av[_D:])
    z_pad, p8 = _dense(x, W1, W2, wp)
    s, ex = _sc_scalar(p8, dst, src)
    res = _sc_rows(z_pad, dst, src, ex)
    return _final(z_pad, res, s[:, None])
